# baseline (speedup 1.0000x reference)
import functools

import jax
import jax.numpy as jnp
from jax import lax
from jax.experimental import pallas as pl
from jax.experimental.pallas import tpu as pltpu
from jax.experimental.pallas import tpu_sc as plsc

NEG = -1e30


def _make_sc_gather(n_rows, D, W):
    NW = 32
    RPW = n_rows // NW
    CH = 16
    NCH = RPW // CH
    GR = CH * W

    mesh = plsc.VectorSubcoreMesh(core_axis_name="c", subcore_axis_name="s")

    @functools.partial(
        pl.kernel,
        mesh=mesh,
        out_type=jax.ShapeDtypeStruct((n_rows, D), jnp.float32),
        scratch_types=[
            pltpu.VMEM((GR,), jnp.int32),
            pltpu.VMEM((GR,), jnp.int32),
            pltpu.VMEM((GR, D), jnp.float32),
            pltpu.VMEM((GR, D), jnp.float32),
            pltpu.VMEM((GR, 16), jnp.float32),
            pltpu.VMEM((GR, 16), jnp.float32),
            pltpu.VMEM((CH, D), jnp.float32),
            pltpu.SemaphoreType.DMA,
            pltpu.SemaphoreType.DMA,
        ],
    )
    def sc_gather(bert_hbm, idx_hbm, msk_hbm, out_hbm, idx_v0, idx_v1,
                  rows_v0, rows_v1, m_v0, m_v1, out_v, sem0, sem1):
        wid = lax.axis_index("s") * 2 + lax.axis_index("c")
        idx_v = (idx_v0, idx_v1)
        rows_v = (rows_v0, rows_v1)
        m_v = (m_v0, m_v1)
        sems = (sem0, sem1)

        def issue(ci, b):
            base_g = (wid * RPW + ci * CH) * W
            pltpu.sync_copy(idx_hbm.at[pl.ds(base_g, GR)], idx_v[b])
            pltpu.async_copy(bert_hbm.at[idx_v[b]], rows_v[b], sems[b])
            pltpu.sync_copy(msk_hbm.at[pl.ds(base_g, GR)], m_v[b])

        def compute(ci, b):
            rv = rows_v[b]
            mv = m_v[b]

            def row_body(j, carry):
                m0 = mv[2 * j]
                m1 = mv[2 * j + 1]
                for c in range(D // 16):
                    s = pl.ds(c * 16, 16)
                    out_v[j, s] = rv[2 * j, s] * m0 + rv[2 * j + 1, s] * m1
                return carry

            lax.fori_loop(0, CH, row_body, 0)
            base_out = wid * RPW + ci * CH
            pltpu.sync_copy(out_v, out_hbm.at[pl.ds(base_out, CH)])

        issue(0, 0)

        def pair_body(g, carry):
            for b in (0, 1):
                ci = 2 * g + b
                nci = jnp.minimum(ci + 1, NCH - 1)
                issue(nci, 1 - b)
                pltpu.make_async_copy(
                    bert_hbm.at[idx_v[b]], rows_v[b], sems[b]).wait()
                compute(ci, b)
            return carry

        lax.fori_loop(0, NCH // 2, pair_body, 0)
        pltpu.make_async_copy(bert_hbm.at[idx_v[0]], rows_v[0], sems[0]).wait()

    return sc_gather


def _make_tc_mlp_crf(B, T, D, H, KP, RT):
    NT = (T * B) // RT

    def body(x_ref, w1_ref, b1_ref, w2_ref, b2_ref, tr_ref, st_ref, en_ref,
             len_ref, out_ref, sc_scr):
        i = pl.program_id(0)
        x = x_ref[...]
        h = jnp.maximum(
            jnp.dot(x, w1_ref[...], preferred_element_type=jnp.float32)
            + b1_ref[...], 0.0)
        s = jnp.dot(h, w2_ref[...], preferred_element_type=jnp.float32) \
            + b2_ref[...]
        cm = jnp.max(s, axis=1, keepdims=True)
        expE = jnp.exp(s - cm)
        lane = lax.broadcasted_iota(jnp.int32, (RT, KP), 1)
        sc_scr[pl.ds(i * RT, RT), :] = jnp.where(lane == KP - 1, cm, expE)

        @pl.when(i == NT - 1)
        def _():
            tr = tr_ref[...]
            c = jnp.max(tr)
            expT = jnp.exp(tr - c)
            lens = len_ref[...]
            stv = st_ref[...]
            stm = jnp.max(stv)
            expSt = jnp.exp(stv - stm)

            e0 = sc_scr[pl.ds(0, B), :]
            cm0 = e0[:, KP - 1:KP]
            v0 = e0 * expSt
            mu0 = jnp.max(v0, axis=1, keepdims=True)
            a0 = v0 / mu0
            s0 = jnp.broadcast_to(cm0 + stm + jnp.log(mu0), (B, KP))

            def step(t, carry):
                a, sacc = carry
                e = sc_scr[pl.ds(t * B, B), :]
                cmt = e[:, KP - 1:KP]
                u = jnp.dot(a, expT, preferred_element_type=jnp.float32)
                v = u * e
                mu = jnp.max(v, axis=1, keepdims=True)
                na = v / mu
                ns = sacc + jnp.log(mu) + cmt + c
                live = t < lens
                return (jnp.where(live, na, a),
                        jnp.where(live, ns, sacc))

            a, sacc = lax.fori_loop(1, T, step, (a0, s0))
            env = en_ref[...]
            em = jnp.max(env)
            w = a * jnp.exp(env - em)
            z = sacc + em + jnp.log(
                jnp.sum(w, axis=1, keepdims=True))
            out_ref[...] = jnp.broadcast_to(z[:, 0:1], (B, KP))

    return body, NT


def kernel(bert_outputs, token_idxs, token_masks, token_nums, W1, b1, W2, b2,
           transitions, start_trans, end_trans):
    B, L, D = bert_outputs.shape
    TW = token_idxs.shape[1]
    W = 2
    T = TW // W
    H = W1.shape[1]
    K = W2.shape[1]
    KP = 128

    bert_flat = bert_outputs.reshape(B * L, D)
    idx = token_idxs.astype(jnp.int32) + (
        jnp.arange(B, dtype=jnp.int32) * L)[:, None]
    idx_tb = idx.reshape(B, T, W).transpose(1, 0, 2).reshape(T * B * W)
    msk_tb = token_masks.reshape(B, T, W).transpose(1, 0, 2).reshape(
        T * B * W)
    msk_b = jnp.broadcast_to(msk_tb[:, None], (T * B * W, 16))

    W2p = jnp.zeros((H, KP), jnp.float32).at[:, :K].set(W2)
    b2p = jnp.zeros((1, KP), jnp.float32).at[0, :K].set(b2)
    trp = jnp.full((KP, KP), NEG, jnp.float32).at[:K, :K].set(transitions)
    stp = jnp.full((1, KP), NEG, jnp.float32).at[0, :K].set(start_trans)
    enp = jnp.full((1, KP), NEG, jnp.float32).at[0, :K].set(end_trans)
    lens = jnp.maximum(token_nums, 1).astype(jnp.int32)
    lens2d = jnp.broadcast_to(lens[:, None], (B, KP))
    b1_2d = b1.reshape(1, H)

    sc_gather = _make_sc_gather(T * B, D, W)
    reprs = sc_gather(bert_flat, idx_tb, msk_b)

    RT = 512
    body, NT = _make_tc_mlp_crf(B, T, D, H, KP, RT)
    out = pl.pallas_call(
        body,
        grid=(NT,),
        in_specs=[
            pl.BlockSpec((RT, D), lambda i: (i, 0)),
            pl.BlockSpec((D, H), lambda i: (0, 0)),
            pl.BlockSpec((1, H), lambda i: (0, 0)),
            pl.BlockSpec((H, KP), lambda i: (0, 0)),
            pl.BlockSpec((1, KP), lambda i: (0, 0)),
            pl.BlockSpec((KP, KP), lambda i: (0, 0)),
            pl.BlockSpec((1, KP), lambda i: (0, 0)),
            pl.BlockSpec((1, KP), lambda i: (0, 0)),
            pl.BlockSpec((B, KP), lambda i: (0, 0)),
        ],
        out_specs=pl.BlockSpec((B, KP), lambda i: (0, 0)),
        out_shape=jax.ShapeDtypeStruct((B, KP), jnp.float32),
        scratch_shapes=[pltpu.VMEM((T * B, KP), jnp.float32)],
        compiler_params=pltpu.CompilerParams(
            dimension_semantics=("arbitrary",)),
    )(reprs, W1, b1_2d, W2p, b2p, trp, stp, enp, lens2d)
    return out[:, 0]


# speedup vs baseline: 1.3057x; 1.3057x over previous
import functools

import jax
import jax.numpy as jnp
from jax import lax
from jax.experimental import pallas as pl
from jax.experimental.pallas import tpu as pltpu
from jax.experimental.pallas import tpu_sc as plsc

NEG = -1e30


def _make_sc_gather(n_rows, D, W):
    NW = 32
    RPW = n_rows // NW
    CH = 16
    NCH = RPW // CH
    GR = CH * W

    mesh = plsc.VectorSubcoreMesh(core_axis_name="c", subcore_axis_name="s")

    @functools.partial(
        pl.kernel,
        mesh=mesh,
        out_type=jax.ShapeDtypeStruct((n_rows, D), jnp.float32),
        scratch_types=[
            pltpu.VMEM((GR,), jnp.int32),
            pltpu.VMEM((GR,), jnp.int32),
            pltpu.VMEM((GR, D), jnp.float32),
            pltpu.VMEM((GR, D), jnp.float32),
            pltpu.VMEM((GR, 16), jnp.float32),
            pltpu.VMEM((GR, 16), jnp.float32),
            pltpu.VMEM((CH, D), jnp.float32),
            pltpu.SemaphoreType.DMA,
            pltpu.SemaphoreType.DMA,
        ],
    )
    def sc_gather(bert_hbm, idx_hbm, msk_hbm, out_hbm, idx_v0, idx_v1,
                  rows_v0, rows_v1, m_v0, m_v1, out_v, sem0, sem1):
        wid = lax.axis_index("s") * 2 + lax.axis_index("c")
        idx_v = (idx_v0, idx_v1)
        rows_v = (rows_v0, rows_v1)
        m_v = (m_v0, m_v1)
        sems = (sem0, sem1)

        def issue(ci, b):
            base_g = (wid * RPW + ci * CH) * W
            pltpu.sync_copy(idx_hbm.at[pl.ds(base_g, GR)], idx_v[b])
            pltpu.async_copy(bert_hbm.at[idx_v[b]], rows_v[b], sems[b])
            pltpu.sync_copy(msk_hbm.at[pl.ds(base_g, GR)], m_v[b])

        def compute(ci, b):
            rv = rows_v[b]
            mv = m_v[b]

            @plsc.parallel_loop(0, CH, unroll=2)
            def row_body(j):
                m0 = mv[2 * j]
                m1 = mv[2 * j + 1]
                for c in range(D // 16):
                    s = pl.ds(c * 16, 16)
                    out_v[j, s] = rv[2 * j, s] * m0 + rv[2 * j + 1, s] * m1

            base_out = wid * RPW + ci * CH
            pltpu.sync_copy(out_v, out_hbm.at[pl.ds(base_out, CH)])

        issue(0, 0)

        def pair_body(g, carry):
            for b in (0, 1):
                ci = 2 * g + b
                nci = jnp.minimum(ci + 1, NCH - 1)
                issue(nci, 1 - b)
                pltpu.make_async_copy(
                    bert_hbm.at[idx_v[b]], rows_v[b], sems[b]).wait()
                compute(ci, b)
            return carry

        lax.fori_loop(0, NCH // 2, pair_body, 0)
        pltpu.make_async_copy(bert_hbm.at[idx_v[0]], rows_v[0], sems[0]).wait()

    return sc_gather


def _make_tc_mlp_crf(B, T, D, H, KP, RT):
    NT = (T * B) // RT

    def body(x_ref, w1_ref, b1_ref, w2_ref, b2_ref, tr_ref, st_ref, en_ref,
             len_ref, out_ref, sc_scr):
        i = pl.program_id(0)
        x = x_ref[...]
        h = jnp.maximum(
            jnp.dot(x, w1_ref[...], preferred_element_type=jnp.float32)
            + b1_ref[...], 0.0)
        s = jnp.dot(h, w2_ref[...], preferred_element_type=jnp.float32) \
            + b2_ref[...]
        cm = jnp.max(s, axis=1, keepdims=True)
        expE = jnp.exp(s - cm)
        lane = lax.broadcasted_iota(jnp.int32, (RT, KP), 1)
        sc_scr[pl.ds(i * RT, RT), :] = jnp.where(lane == KP - 1, cm, expE)

        @pl.when(i == NT - 1)
        def _():
            tr = tr_ref[...]
            c = jnp.max(tr)
            expT = jnp.exp(tr - c)
            lens = len_ref[...]
            stv = st_ref[...]
            stm = jnp.max(stv)
            expSt = jnp.exp(stv - stm)

            e0 = sc_scr[pl.ds(0, B), :]
            cm0 = e0[:, KP - 1:KP]
            v0 = e0 * expSt
            mu0 = jnp.max(v0, axis=1, keepdims=True)
            a0 = v0 / mu0
            s0 = jnp.broadcast_to(cm0 + stm + jnp.log(mu0), (B, KP))
            ones = e0 * 0.0 + 1.0
            zeros = e0 * 0.0

            def step(t, carry):
                w, sacc, r1, lg1, r2, lg2 = carry
                e = sc_scr[pl.ds(t * B, B), :]
                cmt = e[:, KP - 1:KP]
                u = jnp.dot(w, expT, preferred_element_type=jnp.float32)
                v = (u * e) * r2
                ns = sacc + lg2 + cmt + c
                mu = jnp.broadcast_to(
                    jnp.max(v, axis=1, keepdims=True), (B, KP))
                live = t < lens
                return (jnp.where(live, v, w),
                        jnp.where(live, ns, sacc),
                        1.0 / mu, jnp.log(mu), r1, lg1)

            a, sacc, _, _, _, _ = lax.fori_loop(
                1, T, step, (a0, s0, ones, zeros, ones, zeros), unroll=4)
            env = en_ref[...]
            em = jnp.max(env)
            w = a * jnp.exp(env - em)
            z = sacc + em + jnp.log(
                jnp.sum(w, axis=1, keepdims=True))
            out_ref[...] = jnp.broadcast_to(z[:, 0:1], (B, KP))

    return body, NT


def kernel(bert_outputs, token_idxs, token_masks, token_nums, W1, b1, W2, b2,
           transitions, start_trans, end_trans):
    B, L, D = bert_outputs.shape
    TW = token_idxs.shape[1]
    W = 2
    T = TW // W
    H = W1.shape[1]
    K = W2.shape[1]
    KP = 128

    bert_flat = bert_outputs.reshape(B * L, D)
    idx = token_idxs.astype(jnp.int32) + (
        jnp.arange(B, dtype=jnp.int32) * L)[:, None]
    idx_tb = idx.reshape(B, T, W).transpose(1, 0, 2).reshape(T * B * W)
    msk_tb = token_masks.reshape(B, T, W).transpose(1, 0, 2).reshape(
        T * B * W)
    msk_b = jnp.broadcast_to(msk_tb[:, None], (T * B * W, 16))

    W2p = jnp.zeros((H, KP), jnp.float32).at[:, :K].set(W2)
    b2p = jnp.zeros((1, KP), jnp.float32).at[0, :K].set(b2)
    trp = jnp.full((KP, KP), NEG, jnp.float32).at[:K, :K].set(transitions)
    stp = jnp.full((1, KP), NEG, jnp.float32).at[0, :K].set(start_trans)
    enp = jnp.full((1, KP), NEG, jnp.float32).at[0, :K].set(end_trans)
    lens = jnp.maximum(token_nums, 1).astype(jnp.int32)
    lens2d = jnp.broadcast_to(lens[:, None], (B, KP))
    b1_2d = b1.reshape(1, H)

    sc_gather = _make_sc_gather(T * B, D, W)
    reprs = sc_gather(bert_flat, idx_tb, msk_b)

    RT = 512
    body, NT = _make_tc_mlp_crf(B, T, D, H, KP, RT)
    out = pl.pallas_call(
        body,
        grid=(NT,),
        in_specs=[
            pl.BlockSpec((RT, D), lambda i: (i, 0)),
            pl.BlockSpec((D, H), lambda i: (0, 0)),
            pl.BlockSpec((1, H), lambda i: (0, 0)),
            pl.BlockSpec((H, KP), lambda i: (0, 0)),
            pl.BlockSpec((1, KP), lambda i: (0, 0)),
            pl.BlockSpec((KP, KP), lambda i: (0, 0)),
            pl.BlockSpec((1, KP), lambda i: (0, 0)),
            pl.BlockSpec((1, KP), lambda i: (0, 0)),
            pl.BlockSpec((B, KP), lambda i: (0, 0)),
        ],
        out_specs=pl.BlockSpec((B, KP), lambda i: (0, 0)),
        out_shape=jax.ShapeDtypeStruct((B, KP), jnp.float32),
        scratch_shapes=[pltpu.VMEM((T * B, KP), jnp.float32)],
        compiler_params=pltpu.CompilerParams(
            dimension_semantics=("arbitrary",)),
    )(reprs, W1, b1_2d, W2p, b2p, trp, stp, enp, lens2d)
    return out[:, 0]


# speedup vs baseline: 1.4576x; 1.1163x over previous
import functools

import jax
import jax.numpy as jnp
from jax import lax
from jax.experimental import pallas as pl
from jax.experimental.pallas import tpu as pltpu
from jax.experimental.pallas import tpu_sc as plsc

NEG = -1e30


def _make_sc_gather(n_rows, D, W):
    NW = 32
    RPW = n_rows // NW
    CH = 16
    NCH = RPW // CH
    GR = CH * W

    mesh = plsc.VectorSubcoreMesh(core_axis_name="c", subcore_axis_name="s")

    @functools.partial(
        pl.kernel,
        mesh=mesh,
        out_type=jax.ShapeDtypeStruct((n_rows, D), jnp.float32),
        scratch_types=[
            pltpu.VMEM((RPW * W,), jnp.int32),
            pltpu.VMEM((GR, 16), jnp.float32),
            pltpu.VMEM((GR, 16), jnp.float32),
            pltpu.VMEM((GR, D), jnp.float32),
            pltpu.VMEM((GR, D), jnp.float32),
            pltpu.VMEM((CH, D), jnp.float32),
            pltpu.SemaphoreType.DMA,
            pltpu.SemaphoreType.DMA,
            pltpu.SemaphoreType.DMA,
            pltpu.SemaphoreType.DMA,
            pltpu.SemaphoreType.DMA,
        ],
    )
    def sc_gather(bert_hbm, idx_hbm, msk_hbm, out_hbm, idx_v, m_v0, m_v1,
                  rows_v0, rows_v1, out_v, sem0, sem1, msem0, msem1, osem):
        wid = lax.axis_index("s") * 2 + lax.axis_index("c")
        rows_v = (rows_v0, rows_v1)
        m_v = (m_v0, m_v1)
        sems = (sem0, sem1)
        msems = (msem0, msem1)
        base_w = wid * RPW * W

        pltpu.sync_copy(idx_hbm.at[pl.ds(base_w, RPW * W)], idx_v)

        def issue(ci, b):
            pltpu.async_copy(
                bert_hbm.at[idx_v.at[pl.ds(ci * GR, GR)]], rows_v[b], sems[b])
            pltpu.async_copy(
                msk_hbm.at[pl.ds(base_w + ci * GR, GR)], m_v[b], msems[b])

        def compute(ci, b):
            rv = rows_v[b]
            mv = m_v[b]

            @pl.when(ci >= 1)
            def _():
                pltpu.make_async_copy(
                    out_v, out_hbm.at[pl.ds(wid * RPW, CH)], osem).wait()

            @plsc.parallel_loop(0, CH, unroll=2)
            def row_body(j):
                m0 = mv[2 * j]
                m1 = mv[2 * j + 1]
                for c in range(D // 16):
                    s = pl.ds(c * 16, 16)
                    out_v[j, s] = rv[2 * j, s] * m0 + rv[2 * j + 1, s] * m1

            base_out = wid * RPW + ci * CH
            pltpu.async_copy(out_v, out_hbm.at[pl.ds(base_out, CH)], osem)

        issue(0, 0)

        def pair_body(g, carry):
            for b in (0, 1):
                ci = 2 * g + b
                nci = jnp.minimum(ci + 1, NCH - 1)
                issue(nci, 1 - b)
                pltpu.make_async_copy(
                    bert_hbm.at[idx_v.at[pl.ds(0, GR)]], rows_v[b],
                    sems[b]).wait()
                pltpu.make_async_copy(
                    msk_hbm.at[pl.ds(base_w, GR)], m_v[b], msems[b]).wait()
                compute(ci, b)
            return carry

        lax.fori_loop(0, NCH // 2, pair_body, 0)
        pltpu.make_async_copy(
            bert_hbm.at[idx_v.at[pl.ds(0, GR)]], rows_v[0], sems[0]).wait()
        pltpu.make_async_copy(
            msk_hbm.at[pl.ds(base_w, GR)], m_v[0], msems[0]).wait()
        pltpu.make_async_copy(
            out_v, out_hbm.at[pl.ds(wid * RPW, CH)], osem).wait()

    return sc_gather


def _make_tc_mlp_crf(B, T, D, H, KP, RT):
    NT = (T * B) // RT

    def body(x_ref, w1_ref, b1_ref, w2_ref, b2_ref, tr_ref, st_ref, en_ref,
             len_ref, out_ref, sc_scr):
        i = pl.program_id(0)
        x = x_ref[...]
        h = jnp.maximum(
            jnp.dot(x, w1_ref[...], preferred_element_type=jnp.float32)
            + b1_ref[...], 0.0)
        s = jnp.dot(h, w2_ref[...], preferred_element_type=jnp.float32) \
            + b2_ref[...]
        cm = jnp.max(s, axis=1, keepdims=True)
        expE = jnp.exp(s - cm)
        lane = lax.broadcasted_iota(jnp.int32, (RT, KP), 1)
        sc_scr[pl.ds(i * RT, RT), :] = jnp.where(lane == KP - 1, cm, expE)

        @pl.when(i == NT - 1)
        def _():
            tr = tr_ref[...]
            c = jnp.max(tr)
            expT = jnp.exp(tr - c)
            lens = len_ref[...]
            stv = st_ref[...]
            stm = jnp.max(stv)
            expSt = jnp.exp(stv - stm)

            e0 = sc_scr[pl.ds(0, B), :]
            cm0 = e0[:, KP - 1:KP]
            v0 = e0 * expSt
            mu0 = jnp.max(v0, axis=1, keepdims=True)
            a0 = v0 / mu0
            s0 = jnp.broadcast_to(cm0 + stm + jnp.log(mu0), (B, KP))
            ones = e0 * 0.0 + 1.0
            zeros = e0 * 0.0

            def step(t, carry):
                w, sacc, r1, lg1, r2, lg2 = carry
                e = sc_scr[pl.ds(t * B, B), :]
                cmt = e[:, KP - 1:KP]
                u = jnp.dot(w, expT, preferred_element_type=jnp.float32)
                v = (u * e) * r2
                ns = sacc + lg2 + cmt + c
                mu = jnp.broadcast_to(
                    jnp.max(v, axis=1, keepdims=True), (B, KP))
                live = t < lens
                return (jnp.where(live, v, w),
                        jnp.where(live, ns, sacc),
                        1.0 / mu, jnp.log(mu), r1, lg1)

            a, sacc, _, _, _, _ = lax.fori_loop(
                1, T, step, (a0, s0, ones, zeros, ones, zeros), unroll=4)
            env = en_ref[...]
            em = jnp.max(env)
            w = a * jnp.exp(env - em)
            z = sacc + em + jnp.log(
                jnp.sum(w, axis=1, keepdims=True))
            out_ref[...] = jnp.broadcast_to(z[:, 0:1], (B, KP))

    return body, NT


def kernel(bert_outputs, token_idxs, token_masks, token_nums, W1, b1, W2, b2,
           transitions, start_trans, end_trans):
    B, L, D = bert_outputs.shape
    TW = token_idxs.shape[1]
    W = 2
    T = TW // W
    H = W1.shape[1]
    K = W2.shape[1]
    KP = 128

    bert_flat = bert_outputs.reshape(B * L, D)
    idx = token_idxs.astype(jnp.int32) + (
        jnp.arange(B, dtype=jnp.int32) * L)[:, None]
    idx_tb = idx.reshape(B, T, W).transpose(1, 0, 2).reshape(T * B * W)
    msk_tb = token_masks.reshape(B, T, W).transpose(1, 0, 2).reshape(
        T * B * W)
    msk_b = jnp.broadcast_to(msk_tb[:, None], (T * B * W, 16))

    W2p = jnp.zeros((H, KP), jnp.float32).at[:, :K].set(W2)
    b2p = jnp.zeros((1, KP), jnp.float32).at[0, :K].set(b2)
    trp = jnp.full((KP, KP), NEG, jnp.float32).at[:K, :K].set(transitions)
    stp = jnp.full((1, KP), NEG, jnp.float32).at[0, :K].set(start_trans)
    enp = jnp.full((1, KP), NEG, jnp.float32).at[0, :K].set(end_trans)
    lens = jnp.maximum(token_nums, 1).astype(jnp.int32)
    lens2d = jnp.broadcast_to(lens[:, None], (B, KP))
    b1_2d = b1.reshape(1, H)

    sc_gather = _make_sc_gather(T * B, D, W)
    reprs = sc_gather(bert_flat, idx_tb, msk_b)

    RT = 512
    body, NT = _make_tc_mlp_crf(B, T, D, H, KP, RT)
    out = pl.pallas_call(
        body,
        grid=(NT,),
        in_specs=[
            pl.BlockSpec((RT, D), lambda i: (i, 0)),
            pl.BlockSpec((D, H), lambda i: (0, 0)),
            pl.BlockSpec((1, H), lambda i: (0, 0)),
            pl.BlockSpec((H, KP), lambda i: (0, 0)),
            pl.BlockSpec((1, KP), lambda i: (0, 0)),
            pl.BlockSpec((KP, KP), lambda i: (0, 0)),
            pl.BlockSpec((1, KP), lambda i: (0, 0)),
            pl.BlockSpec((1, KP), lambda i: (0, 0)),
            pl.BlockSpec((B, KP), lambda i: (0, 0)),
        ],
        out_specs=pl.BlockSpec((B, KP), lambda i: (0, 0)),
        out_shape=jax.ShapeDtypeStruct((B, KP), jnp.float32),
        scratch_shapes=[pltpu.VMEM((T * B, KP), jnp.float32)],
        compiler_params=pltpu.CompilerParams(
            dimension_semantics=("arbitrary",)),
    )(reprs, W1, b1_2d, W2p, b2p, trp, stp, enp, lens2d)
    return out[:, 0]
